# baseline (speedup 1.0000x reference)
import dataclasses

import jax
import jax.numpy as jnp
from jax import lax
from jax.experimental import pallas as pl
from jax.experimental.pallas import tpu as pltpu
from jax.experimental.pallas import tpu_sc as plsc

B, N, C = 32, 20000, 62
K = 200
L = 16
NCH = N // L
WPAD = 208
NDC = 58
RP = WPAD
OUT_C = 53
OUT_F = K * OUT_C
SCALE = 512.0


def _sc_body(yt2d, out_hbm, conf2, pbuf_a, pbuf_b, cbits, hist, hist16, rows_f,
             wval, widx, tidx, rankv, out_v, sem_c, sem_a, sem_b):
    b = lax.axis_index("s") * 2 + lax.axis_index("c")
    iota = lax.iota(jnp.int32, L)
    zeros_i = jnp.zeros((L,), jnp.int32)
    ones = jnp.full((L,), True)

    zc, _ = plsc.scan_count(zeros_i, mask=ones)
    adj = L - jnp.max(zc)

    with jax.named_scope("conf_dma"):
        pltpu.sync_copy(yt2d.at[pl.ds(1 * B + b, 1), :], conf2)

    def conf(i):
        return conf2[0, pl.ds(i * L, L)]

    def hist_zero(nb):
        @pl.loop(0, nb // L)
        def _(j):
            hist[pl.ds(j * L, L)] = jnp.zeros((L,), jnp.int32)

    def hist_add(bkt, valid):
        cnt, lastm = plsc.scan_count(bkt, mask=valid)
        plsc.addupdate_scatter(hist, [bkt], cnt + adj, mask=lastm)

    def hist_scan(k_lvl, nb):
        m = nb // L
        def body(jj, carry):
            acc, q, n_above, found = carry
            j = m - 1 - jj
            h = hist[pl.ds(j * L, L)]
            s = jnp.sum(h)
            rev = lax.rev(h, (0,))
            tot = acc + plsc.cumsum(rev)
            p = jnp.min(jnp.where(tot >= k_lvl, iota, L))
            here = jnp.logical_and(found == 0, acc + s >= k_lvl)
            p_ = jnp.minimum(p, L - 1)
            qq = j * L + (L - 1) - p_
            above = jnp.sum(jnp.where(iota == p_, tot - rev, 0))
            q = jnp.where(here, qq, q)
            n_above = jnp.where(here, above, n_above)
            found = jnp.where(here, 1, found)
            acc = jnp.where(found == 1, acc, acc + s)
            return acc, q, n_above, found
        _, q, n_above, _ = lax.fori_loop(
            0, m, body, (jnp.int32(0), jnp.int32(0), jnp.int32(0), jnp.int32(0)))
        return q, k_lvl - n_above

    scope_radix = jax.named_scope("radix")
    scope_radix.__enter__()
    one_i = jnp.full((L,), 1, jnp.int32)
    lane_off = iota * 1025
    @pl.loop(0, (L * 1025 + L) // L, unroll=4)
    def _(j):
        hist16[pl.ds(j * L, L)] = jnp.zeros((L,), jnp.int32)
    @pl.loop(0, NCH, unroll=4)
    def _(i):
        bts = plsc.bitcast(conf(i), jnp.int32)
        plsc.addupdate_scatter(hist16, [(bts >> 22) + lane_off], one_i)
    @pl.loop(0, 1024 // L, unroll=2)
    def _(jb):
        acc = hist16[pl.ds(jb * L, L)]
        for l in range(1, L):
            acc = acc + hist16[pl.ds(l * 1025 + jb * L, L)]
        hist[pl.ds(jb * L, L)] = acc
    q0, k1 = hist_scan(jnp.int32(K), 1024)

    def compress0(i, off):
        bts = plsc.bitcast(conf(i), jnp.int32)
        m = (bts >> 22) == q0
        plsc.store_compressed(cbits.at[pl.ds(off, L)], bts, mask=m)
        return off + jnp.sum(m.astype(jnp.int32))
    n_cand = lax.fori_loop(0, NCH, compress0, jnp.int32(0), unroll=2)

    ncc = (n_cand + (L - 1)) // L
    hist_zero(2048)
    def hist1(i, _):
        bts = cbits[pl.ds(i * L, L)]
        hist_add((bts >> 11) & 0x7FF, (i * L + iota) < n_cand)
        return 0
    lax.fori_loop(0, ncc, hist1, 0)
    q1, k2 = hist_scan(k1, 2048)

    def compress1(i, off):
        bts = cbits[pl.ds(i * L, L)]
        m = (((bts >> 11) & 0x7FF) == q1) & ((i * L + iota) < n_cand)
        plsc.store_compressed(cbits.at[pl.ds(off, L)], bts, mask=m)
        return off + jnp.sum(m.astype(jnp.int32))
    n_cand2 = lax.fori_loop(0, ncc, compress1, jnp.int32(0))

    ncc2 = (n_cand2 + (L - 1)) // L
    hist_zero(2048)
    def hist2(i, _):
        bts = cbits[pl.ds(i * L, L)]
        hist_add(bts & 0x7FF, (i * L + iota) < n_cand2)
        return 0
    lax.fori_loop(0, ncc2, hist2, 0)
    q2, k3 = hist_scan(k2, 2048)

    scope_radix.__exit__(None, None, None)
    t_bits = (q0 << 22) | (q1 << 11) | q2
    rem_k = k3
    n_gt = K - rem_k

    scope_collect = jax.named_scope("collect")
    scope_collect.__enter__()
    def collect(i, carry):
        og, ot = carry
        v = conf(i)
        bts = plsc.bitcast(v, jnp.int32)
        m_gt = bts > t_bits
        m_eq = bts == t_bits
        s_gt = jnp.sum(m_gt.astype(jnp.int32))
        s_eq = jnp.sum(m_eq.astype(jnp.int32))
        @pl.when(s_gt > 0)
        def _():
            plsc.store_compressed(wval.at[pl.ds(og, L)], v, mask=m_gt)
            plsc.store_compressed(widx.at[pl.ds(og, L)], i * L + iota, mask=m_gt)
        @pl.when((ot < K) & (s_eq > 0))
        def _():
            plsc.store_compressed(tidx.at[pl.ds(ot, L)], i * L + iota, mask=m_eq)
        return og + s_gt, ot + s_eq
    lax.fori_loop(0, NCH, collect, (jnp.int32(0), jnp.int32(0)), unroll=2)

    t_f = plsc.bitcast(jnp.full((L,), t_bits, jnp.int32), jnp.float32)
    @pl.loop(0, 13)
    def _(j):
        rem = rem_k - j * L
        @pl.when(rem > 0)
        def _():
            tch = tidx[pl.ds(j * L, L)]
            m = iota < rem
            plsc.store_compressed(wval.at[pl.ds(n_gt + j * L, L)], t_f, mask=m)
            plsc.store_compressed(widx.at[pl.ds(n_gt + j * L, L)], tch, mask=m)

    wval[pl.ds(K - 8, L)] = jnp.where(iota < 8, wval[pl.ds(K - 8, L)], -1.0)
    widx[pl.ds(K - 8, L)] = jnp.where(iota < 8, widx[pl.ds(K - 8, L)], iota)

    scope_collect.__exit__(None, None, None)
    scope_planes = jax.named_scope("planes")
    scope_planes.__enter__()
    def firep(cc, buf, sem):
        pltpu.async_copy(yt2d.at[pl.ds((cc + 2) * B + b, 1), :], buf, sem)

    def waitp(cc, buf, sem):
        pltpu.make_async_copy(yt2d.at[pl.ds((cc + 2) * B + b, 1), :], buf,
                              sem).wait()

    def extractp(cc, buf):
        @pl.loop(0, WPAD // L)
        def _(a):
            wch = widx[pl.ds(a * L, L)]
            v = plsc.load_gather(buf, [zeros_i, wch])
            rows_f[pl.ds(cc * RP + a * L, L)] = v

    firep(0, pbuf_a, sem_a)
    firep(1, pbuf_b, sem_b)
    @pl.loop(0, NDC // 2)
    def _(t):
        cc = 2 * t
        waitp(cc, pbuf_a, sem_a)
        extractp(cc, pbuf_a)
        @pl.when(cc + 2 < NDC)
        def _():
            firep(cc + 2, pbuf_a, sem_a)
        waitp(cc + 1, pbuf_b, sem_b)
        extractp(cc + 1, pbuf_b)
        @pl.when(cc + 3 < NDC)
        def _():
            firep(cc + 3, pbuf_b, sem_b)

    scope_planes.__exit__(None, None, None)
    scope_rank = jax.named_scope("rank")
    scope_rank.__enter__()
    @pl.loop(0, WPAD // L)
    def _(a):
        va = wval[pl.ds(a * L, L)]
        ia = widx[pl.ds(a * L, L)]
        def bodyj(j, acc):
            jv = jnp.full((L,), j, jnp.int32)
            vb = plsc.load_gather(wval, [jv])
            ib = plsc.load_gather(widx, [jv])
            ahead = (vb > va) | ((vb == va) & (ib < ia))
            return acc + ahead.astype(jnp.int32)
        rankv[pl.ds(a * L, L)] = lax.fori_loop(0, K, bodyj, jnp.zeros((L,), jnp.int32), unroll=4)

    scope_rank.__exit__(None, None, None)
    scope_decode = jax.named_scope("decode")
    scope_decode.__enter__()
    @pl.loop(0, WPAD // L)
    def _(a):
        r = rankv[pl.ds(a * L, L)]
        valid = r < K
        r53 = r * OUT_C
        sc = wval[pl.ds(a * L, L)]
        plsc.store_scatter(out_v, [r53], sc, mask=valid)
        xy = [rows_f[pl.ds((52 + p) * RP + a * L, L)] for p in (0, 1)]
        wh = [rows_f[pl.ds((54 + p) * RP + a * L, L)] for p in (0, 1)]
        vr = [rows_f[pl.ds((56 + p) * RP + a * L, L)] for p in (0, 1)]
        for c in range(52):
            p = c & 1
            off = rows_f[pl.ds(c * RP + a * L, L)]
            d = ((off * wh[p]) * vr[p] + xy[p]) * SCALE
            plsc.store_scatter(out_v, [r53 + (1 + c)], d, mask=valid)

    scope_decode.__exit__(None, None, None)
    with jax.named_scope("out_dma"):
        pltpu.sync_copy(out_v, out_hbm.at[b])


def kernel(y_pred):
    yt2d = jnp.transpose(y_pred, (2, 0, 1)).reshape(C * B, N)
    mesh = plsc.VectorSubcoreMesh(core_axis_name="c", subcore_axis_name="s")
    cp = pltpu.CompilerParams()
    if "needs_layout_passes" in pltpu.CompilerParams.__dataclass_fields__:
        cp = dataclasses.replace(cp, needs_layout_passes=False)
    if "use_tc_tiling_on_sc" in pltpu.CompilerParams.__dataclass_fields__:
        cp = dataclasses.replace(cp, use_tc_tiling_on_sc=True)
    f = pl.kernel(
        _sc_body,
        out_type=jax.ShapeDtypeStruct((B, OUT_F), jnp.float32),
        mesh=mesh,
        compiler_params=cp,
        scratch_types=[
            pltpu.VMEM((1, N), jnp.float32),
            pltpu.VMEM((1, N), jnp.float32),
            pltpu.VMEM((1, N), jnp.float32),
            pltpu.VMEM((N,), jnp.int32),
            pltpu.VMEM((2048,), jnp.int32),
            pltpu.VMEM((L * 1025 + L,), jnp.int32),
            pltpu.VMEM((NDC * RP,), jnp.float32),
            pltpu.VMEM((WPAD + L,), jnp.float32),
            pltpu.VMEM((WPAD + L,), jnp.int32),
            pltpu.VMEM((WPAD + L,), jnp.int32),
            pltpu.VMEM((WPAD,), jnp.int32),
            pltpu.VMEM((OUT_F,), jnp.float32),
            pltpu.SemaphoreType.DMA,
            pltpu.SemaphoreType.DMA,
            pltpu.SemaphoreType.DMA,
        ],
    )
    return f(yt2d).reshape(B, K, OUT_C)


# speedup vs baseline: 1.0389x; 1.0389x over previous
import dataclasses

import jax
import jax.numpy as jnp
from jax import lax
from jax.experimental import pallas as pl
from jax.experimental.pallas import tpu as pltpu
from jax.experimental.pallas import tpu_sc as plsc

B, N, C = 32, 20000, 62
K = 200
L = 16
NCH = N // L
WPAD = 208
NDC = 58
RP = WPAD
OUT_C = 53
OUT_F = K * OUT_C
SCALE = 512.0


def _sc_body(yt2d, out_hbm, conf2, pbuf_a, pbuf_b, cbits, hist, hist16, rows_f,
             wval, widx, tidx, rankv, out_v, sem_c, sem_a, sem_b):
    b = lax.axis_index("s") * 2 + lax.axis_index("c")
    iota = lax.iota(jnp.int32, L)
    zeros_i = jnp.zeros((L,), jnp.int32)
    ones = jnp.full((L,), True)

    zc, _ = plsc.scan_count(zeros_i, mask=ones)
    adj = L - jnp.max(zc)

    with jax.named_scope("conf_dma"):
        pltpu.sync_copy(yt2d.at[pl.ds(1 * B + b, 1), :], conf2)

    def conf(i):
        return conf2[0, pl.ds(i * L, L)]

    def hist_zero(nb):
        @pl.loop(0, nb // L)
        def _(j):
            hist[pl.ds(j * L, L)] = jnp.zeros((L,), jnp.int32)

    def hist_add(bkt, valid):
        cnt, lastm = plsc.scan_count(bkt, mask=valid)
        plsc.addupdate_scatter(hist, [bkt], cnt + adj, mask=lastm)

    def hist_scan(k_lvl, nb):
        m = nb // L
        def body(jj, carry):
            acc, q, n_above, found = carry
            j = m - 1 - jj
            h = hist[pl.ds(j * L, L)]
            s = jnp.sum(h)
            rev = lax.rev(h, (0,))
            tot = acc + plsc.cumsum(rev)
            p = jnp.min(jnp.where(tot >= k_lvl, iota, L))
            here = jnp.logical_and(found == 0, acc + s >= k_lvl)
            p_ = jnp.minimum(p, L - 1)
            qq = j * L + (L - 1) - p_
            above = jnp.sum(jnp.where(iota == p_, tot - rev, 0))
            q = jnp.where(here, qq, q)
            n_above = jnp.where(here, above, n_above)
            found = jnp.where(here, 1, found)
            acc = jnp.where(found == 1, acc, acc + s)
            return acc, q, n_above, found
        _, q, n_above, _ = lax.fori_loop(
            0, m, body, (jnp.int32(0), jnp.int32(0), jnp.int32(0), jnp.int32(0)))
        return q, k_lvl - n_above

    scope_radix = jax.named_scope("radix")
    scope_radix.__enter__()
    one_i = jnp.full((L,), 1, jnp.int32)
    lane_off = iota * 1025
    @pl.loop(0, (L * 1025 + L) // L, unroll=4)
    def _(j):
        hist16[pl.ds(j * L, L)] = jnp.zeros((L,), jnp.int32)
    @pl.loop(0, NCH, unroll=4)
    def _(i):
        bts = plsc.bitcast(conf(i), jnp.int32)
        plsc.addupdate_scatter(hist16, [(bts >> 22) + lane_off], one_i)
    @pl.loop(0, 1024 // L, unroll=2)
    def _(jb):
        acc = hist16[pl.ds(jb * L, L)]
        for l in range(1, L):
            acc = acc + hist16[pl.ds(l * 1025 + jb * L, L)]
        hist[pl.ds(jb * L, L)] = acc
    q0, k1 = hist_scan(jnp.int32(K), 1024)

    def compress0(i, off):
        bts = plsc.bitcast(conf(i), jnp.int32)
        m = (bts >> 22) == q0
        plsc.store_compressed(cbits.at[pl.ds(off, L)], bts, mask=m)
        return off + jnp.sum(m.astype(jnp.int32))
    n_cand = lax.fori_loop(0, NCH, compress0, jnp.int32(0), unroll=2)

    ncc = (n_cand + (L - 1)) // L
    hist_zero(2048)
    def hist1(i, _):
        bts = cbits[pl.ds(i * L, L)]
        hist_add((bts >> 11) & 0x7FF, (i * L + iota) < n_cand)
        return 0
    lax.fori_loop(0, ncc, hist1, 0)
    q1, k2 = hist_scan(k1, 2048)

    def compress1(i, off):
        bts = cbits[pl.ds(i * L, L)]
        m = (((bts >> 11) & 0x7FF) == q1) & ((i * L + iota) < n_cand)
        plsc.store_compressed(cbits.at[pl.ds(off, L)], bts, mask=m)
        return off + jnp.sum(m.astype(jnp.int32))
    n_cand2 = lax.fori_loop(0, ncc, compress1, jnp.int32(0))

    ncc2 = (n_cand2 + (L - 1)) // L
    hist_zero(2048)
    def hist2(i, _):
        bts = cbits[pl.ds(i * L, L)]
        hist_add(bts & 0x7FF, (i * L + iota) < n_cand2)
        return 0
    lax.fori_loop(0, ncc2, hist2, 0)
    q2, k3 = hist_scan(k2, 2048)

    scope_radix.__exit__(None, None, None)
    t_bits = (q0 << 22) | (q1 << 11) | q2
    rem_k = k3
    n_gt = K - rem_k

    scope_collect = jax.named_scope("collect")
    scope_collect.__enter__()
    def collect(i, carry):
        og, ot = carry
        v = conf(i)
        bts = plsc.bitcast(v, jnp.int32)
        m_gt = bts > t_bits
        m_eq = bts == t_bits
        s_gt = jnp.sum(m_gt.astype(jnp.int32))
        s_eq = jnp.sum(m_eq.astype(jnp.int32))
        @pl.when(s_gt > 0)
        def _():
            plsc.store_compressed(wval.at[pl.ds(og, L)], v, mask=m_gt)
            plsc.store_compressed(widx.at[pl.ds(og, L)], i * L + iota, mask=m_gt)
        @pl.when((ot < K) & (s_eq > 0))
        def _():
            plsc.store_compressed(tidx.at[pl.ds(ot, L)], i * L + iota, mask=m_eq)
        return og + s_gt, ot + s_eq
    lax.fori_loop(0, NCH, collect, (jnp.int32(0), jnp.int32(0)), unroll=2)

    t_f = plsc.bitcast(jnp.full((L,), t_bits, jnp.int32), jnp.float32)
    @pl.loop(0, 13)
    def _(j):
        rem = rem_k - j * L
        @pl.when(rem > 0)
        def _():
            tch = tidx[pl.ds(j * L, L)]
            m = iota < rem
            plsc.store_compressed(wval.at[pl.ds(n_gt + j * L, L)], t_f, mask=m)
            plsc.store_compressed(widx.at[pl.ds(n_gt + j * L, L)], tch, mask=m)

    wval[pl.ds(K - 8, L)] = jnp.where(iota < 8, wval[pl.ds(K - 8, L)], -1.0)
    widx[pl.ds(K - 8, L)] = jnp.where(iota < 8, widx[pl.ds(K - 8, L)], iota)

    scope_collect.__exit__(None, None, None)
    scope_planes = jax.named_scope("planes")
    scope_planes.__enter__()
    def firep(cc, buf, sem):
        pltpu.async_copy(yt2d.at[pl.ds((cc + 2) * B + b, 1), :], buf, sem)

    def waitp(cc, buf, sem):
        pltpu.make_async_copy(yt2d.at[pl.ds((cc + 2) * B + b, 1), :], buf,
                              sem).wait()

    def extractp(cc, buf):
        @pl.loop(0, WPAD // L)
        def _(a):
            wch = widx[pl.ds(a * L, L)]
            v = plsc.load_gather(buf, [zeros_i, wch])
            rows_f[pl.ds(cc * RP + a * L, L)] = v

    bufs = (pbuf_a, pbuf_b, conf2)
    sems = (sem_a, sem_b, sem_c)
    for k in range(3):
        firep(k, bufs[k], sems[k])
    @pl.loop(0, (NDC + 2) // 3)
    def _(t):
        for k in range(3):
            cc = 3 * t + k
            @pl.when(cc < NDC)
            def _():
                waitp(cc, bufs[k], sems[k])
                extractp(cc, bufs[k])
                @pl.when(cc + 3 < NDC)
                def _():
                    firep(cc + 3, bufs[k], sems[k])

    scope_planes.__exit__(None, None, None)
    scope_rank = jax.named_scope("rank")
    scope_rank.__enter__()
    @pl.loop(0, WPAD // L)
    def _(a):
        va = wval[pl.ds(a * L, L)]
        ia = widx[pl.ds(a * L, L)]
        def bodyj(j, acc):
            jv = jnp.full((L,), j, jnp.int32)
            vb = plsc.load_gather(wval, [jv])
            ib = plsc.load_gather(widx, [jv])
            ahead = (vb > va) | ((vb == va) & (ib < ia))
            return acc + ahead.astype(jnp.int32)
        rankv[pl.ds(a * L, L)] = lax.fori_loop(0, K, bodyj, jnp.zeros((L,), jnp.int32), unroll=4)

    scope_rank.__exit__(None, None, None)
    scope_decode = jax.named_scope("decode")
    scope_decode.__enter__()
    @pl.loop(0, WPAD // L)
    def _(a):
        r = rankv[pl.ds(a * L, L)]
        valid = r < K
        r53 = r * OUT_C
        sc = wval[pl.ds(a * L, L)]
        plsc.store_scatter(out_v, [r53], sc, mask=valid)
        xy = [rows_f[pl.ds((52 + p) * RP + a * L, L)] for p in (0, 1)]
        wh = [rows_f[pl.ds((54 + p) * RP + a * L, L)] for p in (0, 1)]
        vr = [rows_f[pl.ds((56 + p) * RP + a * L, L)] for p in (0, 1)]
        for c in range(52):
            p = c & 1
            off = rows_f[pl.ds(c * RP + a * L, L)]
            d = ((off * wh[p]) * vr[p] + xy[p]) * SCALE
            plsc.store_scatter(out_v, [r53 + (1 + c)], d, mask=valid)

    scope_decode.__exit__(None, None, None)
    with jax.named_scope("out_dma"):
        pltpu.sync_copy(out_v, out_hbm.at[b])


def kernel(y_pred):
    yt2d = jnp.transpose(y_pred, (2, 0, 1)).reshape(C * B, N)
    mesh = plsc.VectorSubcoreMesh(core_axis_name="c", subcore_axis_name="s")
    cp = pltpu.CompilerParams()
    if "needs_layout_passes" in pltpu.CompilerParams.__dataclass_fields__:
        cp = dataclasses.replace(cp, needs_layout_passes=False)
    if "use_tc_tiling_on_sc" in pltpu.CompilerParams.__dataclass_fields__:
        cp = dataclasses.replace(cp, use_tc_tiling_on_sc=True)
    f = pl.kernel(
        _sc_body,
        out_type=jax.ShapeDtypeStruct((B, OUT_F), jnp.float32),
        mesh=mesh,
        compiler_params=cp,
        scratch_types=[
            pltpu.VMEM((1, N), jnp.float32),
            pltpu.VMEM((1, N), jnp.float32),
            pltpu.VMEM((1, N), jnp.float32),
            pltpu.VMEM((N,), jnp.int32),
            pltpu.VMEM((2048,), jnp.int32),
            pltpu.VMEM((L * 1025 + L,), jnp.int32),
            pltpu.VMEM((NDC * RP,), jnp.float32),
            pltpu.VMEM((WPAD + L,), jnp.float32),
            pltpu.VMEM((WPAD + L,), jnp.int32),
            pltpu.VMEM((WPAD + L,), jnp.int32),
            pltpu.VMEM((WPAD,), jnp.int32),
            pltpu.VMEM((OUT_F,), jnp.float32),
            pltpu.SemaphoreType.DMA,
            pltpu.SemaphoreType.DMA,
            pltpu.SemaphoreType.DMA,
        ],
    )
    return f(yt2d).reshape(B, K, OUT_C)


# speedup vs baseline: 1.0847x; 1.0441x over previous
import dataclasses

import jax
import jax.numpy as jnp
from jax import lax
from jax.experimental import pallas as pl
from jax.experimental.pallas import tpu as pltpu
from jax.experimental.pallas import tpu_sc as plsc

B, N, C = 32, 20000, 62
K = 200
L = 16
NCH = N // L
WPAD = 208
NDC = 58
RP = WPAD
OUT_C = 53
OUT_F = K * OUT_C
SCALE = 512.0


def _sc_body(yt2d, out_hbm, conf2, pbuf_a, pbuf_b, cbits, cidx, hist, rows_f,
             wval, widx, tidx, rankv, out_v, sem_c, sem_a, sem_b):
    b = lax.axis_index("s") * 2 + lax.axis_index("c")
    iota = lax.iota(jnp.int32, L)
    zeros_i = jnp.zeros((L,), jnp.int32)
    ones = jnp.full((L,), True)

    zc, _ = plsc.scan_count(zeros_i, mask=ones)
    adj = L - jnp.max(zc)

    with jax.named_scope("conf_dma"):
        pltpu.sync_copy(yt2d.at[pl.ds(1 * B + b, 1), :], conf2)

    def conf(i):
        return conf2[0, pl.ds(i * L, L)]

    def hist_zero(nb):
        @pl.loop(0, nb // L)
        def _(j):
            hist[pl.ds(j * L, L)] = jnp.zeros((L,), jnp.int32)

    def hist_add(bkt, valid):
        cnt, lastm = plsc.scan_count(bkt, mask=valid)
        plsc.addupdate_scatter(hist, [bkt], cnt + adj, mask=lastm)

    def hist_scan(k_lvl, nb):
        m = nb // L
        def body(jj, carry):
            acc, q, n_above, found = carry
            j = m - 1 - jj
            h = hist[pl.ds(j * L, L)]
            s = jnp.sum(h)
            rev = lax.rev(h, (0,))
            tot = acc + plsc.cumsum(rev)
            p = jnp.min(jnp.where(tot >= k_lvl, iota, L))
            here = jnp.logical_and(found == 0, acc + s >= k_lvl)
            p_ = jnp.minimum(p, L - 1)
            qq = j * L + (L - 1) - p_
            above = jnp.sum(jnp.where(iota == p_, tot - rev, 0))
            q = jnp.where(here, qq, q)
            n_above = jnp.where(here, above, n_above)
            found = jnp.where(here, 1, found)
            acc = jnp.where(found == 1, acc, acc + s)
            return acc, q, n_above, found
        _, q, n_above, _ = lax.fori_loop(
            0, m, body, (jnp.int32(0), jnp.int32(0), jnp.int32(0), jnp.int32(0)))
        return q, k_lvl - n_above

    scope_radix = jax.named_scope("radix")
    scope_radix.__enter__()
    hist_zero(2048)
    @pl.loop(0, NCH, unroll=4)
    def _(i):
        bts = plsc.bitcast(conf(i), jnp.int32)
        hist_add(bts >> 21, ones)
    q0, k1 = hist_scan(jnp.int32(K), 2048)

    def compress0(i, carry):
        off, og = carry
        v = conf(i)
        bts = plsc.bitcast(v, jnp.int32)
        bkt = bts >> 21
        m = bkt == q0
        m_hi = bkt > q0
        plsc.store_compressed(cbits.at[pl.ds(off, L)], bts, mask=m)
        plsc.store_compressed(cidx.at[pl.ds(off, L)], i * L + iota, mask=m)
        s_hi = jnp.sum(m_hi.astype(jnp.int32))
        @pl.when(s_hi > 0)
        def _():
            plsc.store_compressed(wval.at[pl.ds(og, L)], v, mask=m_hi)
            plsc.store_compressed(widx.at[pl.ds(og, L)], i * L + iota, mask=m_hi)
        return off + jnp.sum(m.astype(jnp.int32)), og + s_hi
    n_cand, n_hi = lax.fori_loop(0, NCH, compress0, (jnp.int32(0), jnp.int32(0)),
                                 unroll=2)

    ncc = (n_cand + (L - 1)) // L
    hist_zero(2048)
    def hist1(i, _):
        bts = cbits[pl.ds(i * L, L)]
        hist_add((bts >> 10) & 0x7FF, (i * L + iota) < n_cand)
        return 0
    lax.fori_loop(0, ncc, hist1, 0)
    q1, k2 = hist_scan(k1, 2048)

    hist_zero(1024)
    def hist2(i, _):
        bts = cbits[pl.ds(i * L, L)]
        m = (((bts >> 10) & 0x7FF) == q1) & ((i * L + iota) < n_cand)
        hist_add(bts & 0x3FF, m)
        return 0
    lax.fori_loop(0, ncc, hist2, 0)
    q2, k3 = hist_scan(k2, 1024)

    scope_radix.__exit__(None, None, None)
    t_bits = (q0 << 21) | (q1 << 10) | q2
    rem_k = k3
    n_gt = K - rem_k

    scope_collect = jax.named_scope("collect")
    scope_collect.__enter__()
    def collect(i, carry):
        og, ot = carry
        valid = (i * L + iota) < n_cand
        bts = cbits[pl.ds(i * L, L)]
        ci = cidx[pl.ds(i * L, L)]
        m_gt = valid & (bts > t_bits)
        m_eq = valid & (bts == t_bits)
        s_gt = jnp.sum(m_gt.astype(jnp.int32))
        s_eq = jnp.sum(m_eq.astype(jnp.int32))
        @pl.when(s_gt > 0)
        def _():
            plsc.store_compressed(wval.at[pl.ds(og, L)],
                                  plsc.bitcast(bts, jnp.float32), mask=m_gt)
            plsc.store_compressed(widx.at[pl.ds(og, L)], ci, mask=m_gt)
        @pl.when((ot < K) & (s_eq > 0))
        def _():
            plsc.store_compressed(tidx.at[pl.ds(ot, L)], ci, mask=m_eq)
        return og + s_gt, ot + s_eq
    lax.fori_loop(0, ncc, collect, (n_hi, jnp.int32(0)))
    scope_collect.__exit__(None, None, None)

    t_f = plsc.bitcast(jnp.full((L,), t_bits, jnp.int32), jnp.float32)
    @pl.loop(0, 13)
    def _(j):
        rem = rem_k - j * L
        @pl.when(rem > 0)
        def _():
            tch = tidx[pl.ds(j * L, L)]
            m = iota < rem
            plsc.store_compressed(wval.at[pl.ds(n_gt + j * L, L)], t_f, mask=m)
            plsc.store_compressed(widx.at[pl.ds(n_gt + j * L, L)], tch, mask=m)

    wval[pl.ds(K - 8, L)] = jnp.where(iota < 8, wval[pl.ds(K - 8, L)], -1.0)
    widx[pl.ds(K - 8, L)] = jnp.where(iota < 8, widx[pl.ds(K - 8, L)], iota)

    scope_collect.__exit__(None, None, None)
    scope_planes = jax.named_scope("planes")
    scope_planes.__enter__()
    def firep(cc, buf, sem):
        pltpu.async_copy(yt2d.at[pl.ds((cc + 2) * B + b, 1), :], buf, sem)

    def waitp(cc, buf, sem):
        pltpu.make_async_copy(yt2d.at[pl.ds((cc + 2) * B + b, 1), :], buf,
                              sem).wait()

    def extractp(cc, buf):
        @pl.loop(0, WPAD // L)
        def _(a):
            wch = widx[pl.ds(a * L, L)]
            v = plsc.load_gather(buf, [zeros_i, wch])
            rows_f[pl.ds(cc * RP + a * L, L)] = v

    bufs = (pbuf_a, pbuf_b, conf2)
    sems = (sem_a, sem_b, sem_c)
    for k in range(3):
        firep(k, bufs[k], sems[k])
    @pl.loop(0, (NDC + 2) // 3)
    def _(t):
        for k in range(3):
            cc = 3 * t + k
            @pl.when(cc < NDC)
            def _():
                waitp(cc, bufs[k], sems[k])
                extractp(cc, bufs[k])
                @pl.when(cc + 3 < NDC)
                def _():
                    firep(cc + 3, bufs[k], sems[k])

    scope_planes.__exit__(None, None, None)
    scope_rank = jax.named_scope("rank")
    scope_rank.__enter__()
    @pl.loop(0, WPAD // L)
    def _(a):
        va = wval[pl.ds(a * L, L)]
        ia = widx[pl.ds(a * L, L)]
        def bodyj(j, acc):
            jv = jnp.full((L,), j, jnp.int32)
            vb = plsc.load_gather(wval, [jv])
            ib = plsc.load_gather(widx, [jv])
            ahead = (vb > va) | ((vb == va) & (ib < ia))
            return acc + ahead.astype(jnp.int32)
        rankv[pl.ds(a * L, L)] = lax.fori_loop(0, K, bodyj, jnp.zeros((L,), jnp.int32), unroll=4)

    scope_rank.__exit__(None, None, None)
    scope_decode = jax.named_scope("decode")
    scope_decode.__enter__()
    @pl.loop(0, WPAD // L)
    def _(a):
        r = rankv[pl.ds(a * L, L)]
        valid = r < K
        r53 = r * OUT_C
        sc = wval[pl.ds(a * L, L)]
        plsc.store_scatter(out_v, [r53], sc, mask=valid)
        xy = [rows_f[pl.ds((52 + p) * RP + a * L, L)] for p in (0, 1)]
        wh = [rows_f[pl.ds((54 + p) * RP + a * L, L)] for p in (0, 1)]
        vr = [rows_f[pl.ds((56 + p) * RP + a * L, L)] for p in (0, 1)]
        for c in range(52):
            p = c & 1
            off = rows_f[pl.ds(c * RP + a * L, L)]
            d = ((off * wh[p]) * vr[p] + xy[p]) * SCALE
            plsc.store_scatter(out_v, [r53 + (1 + c)], d, mask=valid)

    scope_decode.__exit__(None, None, None)
    with jax.named_scope("out_dma"):
        pltpu.sync_copy(out_v, out_hbm.at[b])


def kernel(y_pred):
    yt2d = jnp.transpose(y_pred, (2, 0, 1)).reshape(C * B, N)
    mesh = plsc.VectorSubcoreMesh(core_axis_name="c", subcore_axis_name="s")
    cp = pltpu.CompilerParams()
    if "needs_layout_passes" in pltpu.CompilerParams.__dataclass_fields__:
        cp = dataclasses.replace(cp, needs_layout_passes=False)
    if "use_tc_tiling_on_sc" in pltpu.CompilerParams.__dataclass_fields__:
        cp = dataclasses.replace(cp, use_tc_tiling_on_sc=True)
    f = pl.kernel(
        _sc_body,
        out_type=jax.ShapeDtypeStruct((B, OUT_F), jnp.float32),
        mesh=mesh,
        compiler_params=cp,
        scratch_types=[
            pltpu.VMEM((1, N), jnp.float32),
            pltpu.VMEM((1, N), jnp.float32),
            pltpu.VMEM((1, N), jnp.float32),
            pltpu.VMEM((N,), jnp.int32),
            pltpu.VMEM((N,), jnp.int32),
            pltpu.VMEM((2048,), jnp.int32),
            pltpu.VMEM((NDC * RP,), jnp.float32),
            pltpu.VMEM((WPAD + L,), jnp.float32),
            pltpu.VMEM((WPAD + L,), jnp.int32),
            pltpu.VMEM((WPAD + L,), jnp.int32),
            pltpu.VMEM((WPAD,), jnp.int32),
            pltpu.VMEM((OUT_F,), jnp.float32),
            pltpu.SemaphoreType.DMA,
            pltpu.SemaphoreType.DMA,
            pltpu.SemaphoreType.DMA,
        ],
    )
    return f(yt2d).reshape(B, K, OUT_C)
